# baseline (speedup 1.0000x reference)
import functools

import jax
import jax.numpy as jnp
from jax import lax
from jax.experimental import pallas as pl
from jax.experimental.pallas import tpu as pltpu
from jax.experimental.pallas import tpu_sc as plsc

EMBEDDING_DIM = 64
NUM_CORES = 2
NUM_SUBCORES = 16
NUM_WORKERS = NUM_CORES * NUM_SUBCORES

_mesh = plsc.VectorSubcoreMesh(core_axis_name="c", subcore_axis_name="s")


def _make_gather(batch: int, dim: int, chunk: int):
    assert batch % (NUM_WORKERS * chunk) == 0
    b_per_w = batch // NUM_WORKERS
    n_chunks = b_per_w // chunk

    @functools.partial(
        pl.kernel,
        mesh=_mesh,
        out_type=jax.ShapeDtypeStruct((batch, dim), jnp.float32),
        scratch_types=[
            pltpu.VMEM((chunk,), jnp.int32),
            pltpu.VMEM((chunk, dim), jnp.float32),
            pltpu.SemaphoreType.DMA,
        ],
        compiler_params=pltpu.CompilerParams(use_tc_tiling_on_sc=False),
    )
    def gather_kernel(idx_hbm, table_hbm, out_hbm, idx_v, rows_v, sem):
        wid = lax.axis_index("s") * jnp.int32(NUM_CORES) + lax.axis_index("c")
        wbase = wid * jnp.int32(b_per_w)

        def body(c, carry):
            base = pl.multiple_of(wbase + c * jnp.int32(chunk), chunk)
            pltpu.sync_copy(idx_hbm.at[pl.ds(base, chunk)], idx_v)
            pltpu.async_copy(table_hbm.at[idx_v], rows_v, sem).wait()
            pltpu.sync_copy(rows_v, out_hbm.at[pl.ds(base, chunk)])
            return carry

        lax.fori_loop(jnp.int32(0), jnp.int32(n_chunks), body, jnp.int32(0))

    return gather_kernel


def kernel(keys, table):
    flat = keys.reshape(-1).astype(jnp.int32)
    batch = flat.shape[0]
    out = _make_gather(batch, EMBEDDING_DIM, 512)(flat, table)
    return out.reshape(keys.shape + (EMBEDDING_DIM,))


# speedup vs baseline: 1.0309x; 1.0309x over previous
import functools

import jax
import jax.numpy as jnp
from jax import lax
from jax.experimental import pallas as pl
from jax.experimental.pallas import tpu as pltpu
from jax.experimental.pallas import tpu_sc as plsc

EMBEDDING_DIM = 64
NUM_CORES = 2
NUM_SUBCORES = 16
NUM_WORKERS = NUM_CORES * NUM_SUBCORES

NBUF = 4
K = 2

_mesh = plsc.VectorSubcoreMesh(core_axis_name="c", subcore_axis_name="s")


def _make_gather(batch: int, dim: int, chunk: int):
    assert batch % (NUM_WORKERS * chunk) == 0
    b_per_w = batch // NUM_WORKERS
    n_chunks = b_per_w // chunk
    n_iters = n_chunks + K
    n_outer = -(-n_iters // NBUF)

    @functools.partial(
        pl.kernel,
        mesh=_mesh,
        out_type=jax.ShapeDtypeStruct((batch, dim), jnp.float32),
        scratch_types=[
            pltpu.VMEM((b_per_w,), jnp.int32),
            [pltpu.VMEM((chunk, dim), jnp.float32) for _ in range(NBUF)],
            [pltpu.SemaphoreType.DMA for _ in range(NBUF)],
            [pltpu.SemaphoreType.DMA for _ in range(NBUF)],
        ],
        compiler_params=pltpu.CompilerParams(use_tc_tiling_on_sc=False),
    )
    def gather_kernel(idx_hbm, table_hbm, out_hbm, idx_v, rows, gsem, wsem):
        wid = lax.axis_index("s") * jnp.int32(NUM_CORES) + lax.axis_index("c")
        wbase = pl.multiple_of(wid * jnp.int32(b_per_w), chunk)
        pltpu.sync_copy(idx_hbm.at[pl.ds(wbase, b_per_w)], idx_v)

        def outer(c, carry):
            for b in range(NBUF):
                g = c * jnp.int32(NBUF) + jnp.int32(b)
                @pl.when(jnp.logical_and(g >= NBUF, g < n_chunks))
                def _():
                    pltpu.make_async_copy(
                        rows[b],
                        out_hbm.at[pl.ds(wbase, chunk)],
                        wsem[b],
                    ).wait()

                @pl.when(g < n_chunks)
                def _():
                    off = pl.multiple_of(g * jnp.int32(chunk), chunk)
                    pltpu.make_async_copy(
                        table_hbm.at[idx_v.at[pl.ds(off, chunk)]],
                        rows[b],
                        gsem[b],
                    ).start()

                h = g - jnp.int32(K)
                bh = (b - K) % NBUF

                @pl.when(jnp.logical_and(h >= 0, h < n_chunks))
                def _():
                    pltpu.make_async_copy(
                        table_hbm.at[idx_v.at[pl.ds(jnp.int32(0), chunk)]],
                        rows[bh],
                        gsem[bh],
                    ).wait()
                    hoff = pl.multiple_of(wbase + h * jnp.int32(chunk), chunk)
                    pltpu.make_async_copy(
                        rows[bh],
                        out_hbm.at[pl.ds(hoff, chunk)],
                        wsem[bh],
                    ).start()

            return carry

        lax.fori_loop(jnp.int32(0), jnp.int32(n_outer), outer, jnp.int32(0))

        for b in range(NBUF):
            pltpu.make_async_copy(
                rows[b],
                out_hbm.at[pl.ds(wbase, chunk)],
                wsem[b],
            ).wait()

    return gather_kernel


def kernel(keys, table):
    flat = keys.reshape(-1).astype(jnp.int32)
    batch = flat.shape[0]
    out = _make_gather(batch, EMBEDDING_DIM, 416)(flat, table)
    return out.reshape(keys.shape + (EMBEDDING_DIM,))
